# baseline (speedup 1.0000x reference)
import functools

import jax
import jax.numpy as jnp
import numpy as np
from jax import lax
from jax.experimental import pallas as pl
from jax.experimental.pallas import tpu as pltpu
from jax.experimental.pallas import tpu_sc as plsc

_N_ENT = 100000
_N_REL = 100000
_DIM = 64
_N_EMB = 8
_B = 16384
_B2 = 2 * _B

_NC = 2
_NS = 16
_L = 16
_NW = _NC * _NS
_CPW = _B2 // _NW
_NB = 128
_NSTEP = (_CPW // _NB) * _N_EMB

_PSI = 1.2
_MARGIN = 1.0
_LAMBDA_POS = 2.0
_LAMBDA_NEG = 2.0


def _sc_body(ent_hbm, rel_hbm, hidx_hbm, tidx_hbm, ridx_hbm, out_hbm,
             raw_h, raw_t, raw_r, ix_h, ix_t, ix_r, bh, bt, br, ssq, sem):
  wid = lax.axis_index("s") * _NC + lax.axis_index("c")
  wbase = wid * _CPW

  pltpu.sync_copy(hidx_hbm.at[pl.ds(wbase, _CPW)], raw_h)
  pltpu.sync_copy(tidx_hbm.at[pl.ds(wbase, _CPW)], raw_t)
  pltpu.sync_copy(ridx_hbm.at[pl.ds(wbase, _CPW)], raw_r)

  iota = lax.broadcasted_iota(jnp.int32, (_L,), 0)

  def step_body(s, _):
    blk = s // _N_EMB
    k = s % _N_EMB
    base = blk * _NB
    koff = k * _N_ENT
    for c in range(_NB // _L):
      sl = pl.ds(c * _L, _L)
      bsl = pl.ds(base + c * _L, _L)
      ix_h[sl] = raw_h[bsl] + koff
      ix_t[sl] = raw_t[bsl] + koff
      ix_r[sl] = raw_r[bsl] + koff
    cp_h = pltpu.async_copy(ent_hbm.at[ix_h], bh, sem)
    cp_t = pltpu.async_copy(ent_hbm.at[ix_t], bt, sem)
    cp_r = pltpu.async_copy(rel_hbm.at[ix_r], br, sem)
    cp_h.wait()
    cp_t.wait()
    cp_r.wait()

    f = k % 4
    p_d = jnp.full((_L,), f, jnp.int32) == 3
    sh = jnp.where(f == 2, -1.0, 1.0)
    st = jnp.where(f == 0, -1.0, 1.0)
    sr = jnp.where(f == 1, -1.0, 1.0)

    def group_body(g, _):
      rows = g * _L + iota
      acc = jnp.zeros((_L,), jnp.float32)
      for dd in range(_DIM):
        col = jnp.full((_L,), dd, jnp.int32)
        hv = plsc.load_gather(bh, [rows, col])
        tv = plsc.load_gather(bt, [rows, col])
        rv = plsc.load_gather(br, [rows, col])
        lin = sh * hv + st * tv + sr * rv
        vd = hv - rv * tv
        v = jnp.where(p_d, vd, lin)
        acc = acc + v * v
      ssq[pl.ds(g * _L, _L)] = acc
      return 0

    lax.fori_loop(0, _NB // _L, group_body, 0, unroll=1)
    pltpu.sync_copy(ssq, out_hbm.at[k, pl.ds(wbase + base, _NB)])
    return 0

  lax.fori_loop(0, _NSTEP, step_body, 0, unroll=1)


_sc_ssq = functools.partial(
    pl.kernel,
    out_type=jax.ShapeDtypeStruct((_N_EMB, _B2), jnp.float32),
    mesh=plsc.VectorSubcoreMesh(
        core_axis_name="c", subcore_axis_name="s",
        num_cores=_NC, num_subcores=_NS),
    compiler_params=pltpu.CompilerParams(
        needs_layout_passes=False, use_tc_tiling_on_sc=False),
    scratch_types=[
        pltpu.VMEM((_CPW,), jnp.int32),
        pltpu.VMEM((_CPW,), jnp.int32),
        pltpu.VMEM((_CPW,), jnp.int32),
        pltpu.VMEM((_NB,), jnp.int32),
        pltpu.VMEM((_NB,), jnp.int32),
        pltpu.VMEM((_NB,), jnp.int32),
        pltpu.VMEM((_NB, _DIM), jnp.float32),
        pltpu.VMEM((_NB, _DIM), jnp.float32),
        pltpu.VMEM((_NB, _DIM), jnp.float32),
        pltpu.VMEM((_NB,), jnp.float32),
        pltpu.SemaphoreType.DMA,
    ],
)(_sc_body)


_W = (1.5 / 18.0, 3.0 / 18.0, 1.5 / 18.0, 3.0 / 18.0,
      1.5 / 18.0, 3.0 / 18.0, 1.5 / 18.0, 3.0 / 18.0)


def _tc_body(ssq_ref, loss_ref, pos_ref, neg_ref):
  n = jnp.sqrt(ssq_ref[...])
  score = _W[0] * n[0:1, :]
  for k in range(1, _N_EMB):
    score = score + _W[k] * n[k:k + 1, :]
  score = score - _PSI
  pos = jnp.sum(jnp.maximum(score[:, :_B] - (_LAMBDA_POS - _MARGIN), 0.0))
  neg = jnp.sum(jnp.maximum((_LAMBDA_NEG + _MARGIN) - score[:, _B:], 0.0))
  loss_ref[...] = jnp.full((1, 1), pos + neg, jnp.float32)
  pos_ref[...] = jnp.full((1, 1), pos, jnp.float32)
  neg_ref[...] = jnp.full((1, 1), neg, jnp.float32)


def _tc_finish(ssq):
  return pl.pallas_call(
      _tc_body,
      out_shape=(
          jax.ShapeDtypeStruct((1, 1), jnp.float32),
          jax.ShapeDtypeStruct((1, 1), jnp.float32),
          jax.ShapeDtypeStruct((1, 1), jnp.float32),
      ),
  )(ssq)


def kernel(x_train, x_train_negative, entity_emb, relation_emb):
  ent = entity_emb.reshape(_N_EMB * _N_ENT, _DIM)
  rel = relation_emb.reshape(_N_EMB * _N_REL, _DIM)
  hidx = jnp.concatenate([x_train[:, 0], x_train_negative[:, 0]]).astype(jnp.int32)
  tidx = jnp.concatenate([x_train[:, 1], x_train_negative[:, 1]]).astype(jnp.int32)
  ridx = jnp.concatenate([x_train[:, 2], x_train_negative[:, 2]]).astype(jnp.int32)
  ssq = _sc_ssq(ent, rel, hidx, tidx, ridx)
  loss, pos, neg = _tc_finish(ssq)
  return (loss[0, 0], pos[0, 0], neg[0, 0])


# speedup vs baseline: 1.7142x; 1.7142x over previous
import functools

import jax
import jax.numpy as jnp
import numpy as np
from jax import lax
from jax.experimental import pallas as pl
from jax.experimental.pallas import tpu as pltpu
from jax.experimental.pallas import tpu_sc as plsc

_N_ENT = 100000
_N_REL = 100000
_DIM = 64
_N_EMB = 8
_B = 16384
_B2 = 2 * _B

_NC = 2
_NS = 16
_L = 16
_NW = _NC * _NS
_CPW = _B2 // _NW
_NB = 128
_NSTEP = (_CPW // _NB) * _N_EMB

_PSI = 1.2
_MARGIN = 1.0
_LAMBDA_POS = 2.0
_LAMBDA_NEG = 2.0


def _sc_body(ent_hbm, rel_hbm, hidx_hbm, tidx_hbm, ridx_hbm, out_hbm,
             raw_h, raw_t, raw_r,
             ix_h_a, ix_t_a, ix_r_a, ix_h_b, ix_t_b, ix_r_b,
             bh_a, bt_a, br_a, bh_b, bt_b, br_b,
             ssq_a, ssq_b, sem_a, sem_b, sem_oa, sem_ob):
  wid = lax.axis_index("s") * _NC + lax.axis_index("c")
  wbase = wid * _CPW

  pltpu.sync_copy(hidx_hbm.at[pl.ds(wbase, _CPW)], raw_h)
  pltpu.sync_copy(tidx_hbm.at[pl.ds(wbase, _CPW)], raw_t)
  pltpu.sync_copy(ridx_hbm.at[pl.ds(wbase, _CPW)], raw_r)

  iota = lax.broadcasted_iota(jnp.int32, (_L,), 0)

  def build_and_fire(s, ix_h, ix_t, ix_r, bh, bt, br, sem):
    blk = s // _N_EMB
    k = s % _N_EMB
    base = blk * _NB
    koff = k * _N_ENT
    for c in range(_NB // _L):
      sl = pl.ds(c * _L, _L)
      bsl = pl.ds(base + c * _L, _L)
      ix_h[sl] = raw_h[bsl] + koff
      ix_t[sl] = raw_t[bsl] + koff
      ix_r[sl] = raw_r[bsl] + koff
    pltpu.async_copy(ent_hbm.at[ix_h], bh, sem)
    pltpu.async_copy(ent_hbm.at[ix_t], bt, sem)
    pltpu.async_copy(rel_hbm.at[ix_r], br, sem)

  def compute(s, m, ix_h, ix_t, ix_r, bh, bt, br, sem, ssq, sem_o):
    blk = s // _N_EMB
    k = s % _N_EMB
    base = blk * _NB
    pltpu.make_async_copy(ent_hbm.at[ix_h], bh, sem).wait()
    pltpu.make_async_copy(ent_hbm.at[ix_t], bt, sem).wait()
    pltpu.make_async_copy(rel_hbm.at[ix_r], br, sem).wait()
    dst = out_hbm.at[k, pl.ds(wbase + base, _NB)]

    @pl.when(m > 0)
    def _():
      pltpu.make_async_copy(ssq, dst, sem_o).wait()

    def group_loop(comb):
      def group_body(g, _):
        rows = g * _L + iota
        acc = jnp.zeros((_L,), jnp.float32)
        for dd in range(_DIM):
          col = (iota + dd) & (_DIM - 1)
          hv = plsc.load_gather(bh, [rows, col])
          tv = plsc.load_gather(bt, [rows, col])
          rv = plsc.load_gather(br, [rows, col])
          v = comb(hv, tv, rv)
          acc = acc + v * v
        ssq[pl.ds(g * _L, _L)] = acc
        return 0
      lax.fori_loop(0, _NB // _L, group_body, 0, unroll=1)

    f = k % 4

    @pl.when(f == 0)
    def _():
      group_loop(lambda h, t, r: h + r - t)

    @pl.when(f == 1)
    def _():
      group_loop(lambda h, t, r: h + t - r)

    @pl.when(f == 2)
    def _():
      group_loop(lambda h, t, r: t + r - h)

    @pl.when(f == 3)
    def _():
      group_loop(lambda h, t, r: h - r * t)

    pltpu.async_copy(ssq, dst, sem_o)

  build_and_fire(0, ix_h_a, ix_t_a, ix_r_a, bh_a, bt_a, br_a, sem_a)

  def macro_body(m, _):
    s_a = 2 * m
    s_b = 2 * m + 1
    build_and_fire(s_b, ix_h_b, ix_t_b, ix_r_b, bh_b, bt_b, br_b, sem_b)
    compute(s_a, m, ix_h_a, ix_t_a, ix_r_a, bh_a, bt_a, br_a, sem_a,
            ssq_a, sem_oa)

    @pl.when(m < _NSTEP // 2 - 1)
    def _():
      build_and_fire(s_a + 2, ix_h_a, ix_t_a, ix_r_a, bh_a, bt_a, br_a,
                     sem_a)

    compute(s_b, m, ix_h_b, ix_t_b, ix_r_b, bh_b, bt_b, br_b, sem_b,
            ssq_b, sem_ob)
    return 0

  lax.fori_loop(0, _NSTEP // 2, macro_body, 0, unroll=1)

  last = _NSTEP - 2
  pltpu.make_async_copy(
      ssq_a, out_hbm.at[last % _N_EMB,
                        pl.ds(wbase + (last // _N_EMB) * _NB, _NB)],
      sem_oa).wait()
  last = _NSTEP - 1
  pltpu.make_async_copy(
      ssq_b, out_hbm.at[last % _N_EMB,
                        pl.ds(wbase + (last // _N_EMB) * _NB, _NB)],
      sem_ob).wait()


_sc_ssq = functools.partial(
    pl.kernel,
    out_type=jax.ShapeDtypeStruct((_N_EMB, _B2), jnp.float32),
    mesh=plsc.VectorSubcoreMesh(
        core_axis_name="c", subcore_axis_name="s",
        num_cores=_NC, num_subcores=_NS),
    compiler_params=pltpu.CompilerParams(
        needs_layout_passes=False, use_tc_tiling_on_sc=False),
    scratch_types=[
        pltpu.VMEM((_CPW,), jnp.int32),
        pltpu.VMEM((_CPW,), jnp.int32),
        pltpu.VMEM((_CPW,), jnp.int32),
        pltpu.VMEM((_NB,), jnp.int32),
        pltpu.VMEM((_NB,), jnp.int32),
        pltpu.VMEM((_NB,), jnp.int32),
        pltpu.VMEM((_NB,), jnp.int32),
        pltpu.VMEM((_NB,), jnp.int32),
        pltpu.VMEM((_NB,), jnp.int32),
        pltpu.VMEM((_NB, _DIM), jnp.float32),
        pltpu.VMEM((_NB, _DIM), jnp.float32),
        pltpu.VMEM((_NB, _DIM), jnp.float32),
        pltpu.VMEM((_NB, _DIM), jnp.float32),
        pltpu.VMEM((_NB, _DIM), jnp.float32),
        pltpu.VMEM((_NB, _DIM), jnp.float32),
        pltpu.VMEM((_NB,), jnp.float32),
        pltpu.VMEM((_NB,), jnp.float32),
        pltpu.SemaphoreType.DMA,
        pltpu.SemaphoreType.DMA,
        pltpu.SemaphoreType.DMA,
        pltpu.SemaphoreType.DMA,
    ],
)(_sc_body)


_W = (1.5 / 18.0, 3.0 / 18.0, 1.5 / 18.0, 3.0 / 18.0,
      1.5 / 18.0, 3.0 / 18.0, 1.5 / 18.0, 3.0 / 18.0)


def _tc_body(ssq_ref, loss_ref, pos_ref, neg_ref):
  n = jnp.sqrt(ssq_ref[...])
  score = _W[0] * n[0:1, :]
  for k in range(1, _N_EMB):
    score = score + _W[k] * n[k:k + 1, :]
  score = score - _PSI
  pos = jnp.sum(jnp.maximum(score[:, :_B] - (_LAMBDA_POS - _MARGIN), 0.0))
  neg = jnp.sum(jnp.maximum((_LAMBDA_NEG + _MARGIN) - score[:, _B:], 0.0))
  loss_ref[...] = jnp.full((1, 1), pos + neg, jnp.float32)
  pos_ref[...] = jnp.full((1, 1), pos, jnp.float32)
  neg_ref[...] = jnp.full((1, 1), neg, jnp.float32)


def _tc_finish(ssq):
  return pl.pallas_call(
      _tc_body,
      out_shape=(
          jax.ShapeDtypeStruct((1, 1), jnp.float32),
          jax.ShapeDtypeStruct((1, 1), jnp.float32),
          jax.ShapeDtypeStruct((1, 1), jnp.float32),
      ),
  )(ssq)


def kernel(x_train, x_train_negative, entity_emb, relation_emb):
  ent = entity_emb.reshape(_N_EMB * _N_ENT, _DIM)
  rel = relation_emb.reshape(_N_EMB * _N_REL, _DIM)
  hidx = jnp.concatenate([x_train[:, 0], x_train_negative[:, 0]]).astype(jnp.int32)
  tidx = jnp.concatenate([x_train[:, 1], x_train_negative[:, 1]]).astype(jnp.int32)
  ridx = jnp.concatenate([x_train[:, 2], x_train_negative[:, 2]]).astype(jnp.int32)
  ssq = _sc_ssq(ent, rel, hidx, tidx, ridx)
  loss, pos, neg = _tc_finish(ssq)
  return (loss[0, 0], pos[0, 0], neg[0, 0])


# speedup vs baseline: 3.5771x; 2.0867x over previous
import functools

import jax
import jax.numpy as jnp
import numpy as np
from jax import lax
from jax.experimental import pallas as pl
from jax.experimental.pallas import tpu as pltpu
from jax.experimental.pallas import tpu_sc as plsc

_N_ENT = 100000
_DIM = 64
_N_EMB = 8
_B = 16384
_B2 = 2 * _B

_NC = 2
_NS = 16
_L = 16
_NW = _NC * _NS
_CPW = _B2 // _NW
_NT = 32
_NSTEP = _CPW // _NT
_KK = _N_EMB // 2

_PSI = 1.2
_MARGIN = 1.0
_LAMBDA_POS = 2.0
_LAMBDA_NEG = 2.0

_REPACK_E = 512


def _repack_body(xe_ref, xr_ref, oe_ref, or_ref):
  for x_ref, o_ref in ((xe_ref, oe_ref), (xr_ref, or_ref)):
    x = x_ref[...]
    for kk in range(_KK):
      a = x[2 * kk].T
      b = x[2 * kk + 1].T
      o_ref[kk] = jnp.concatenate([a, b], axis=1)


def _repack(ent_t, rel_t):
  n = (_N_ENT + _REPACK_E - 1) // _REPACK_E
  return pl.pallas_call(
      _repack_body,
      grid=(n,),
      in_specs=[
          pl.BlockSpec((_N_EMB, _DIM, _REPACK_E), lambda i: (0, 0, i)),
          pl.BlockSpec((_N_EMB, _DIM, _REPACK_E), lambda i: (0, 0, i)),
      ],
      out_specs=[
          pl.BlockSpec((_KK, _REPACK_E, 2 * _DIM), lambda i: (0, i, 0)),
          pl.BlockSpec((_KK, _REPACK_E, 2 * _DIM), lambda i: (0, i, 0)),
      ],
      out_shape=[
          jax.ShapeDtypeStruct((_KK, _N_ENT, 2 * _DIM), jnp.float32),
          jax.ShapeDtypeStruct((_KK, _N_ENT, 2 * _DIM), jnp.float32),
      ],
  )(ent_t, rel_t)


def _sc_body(ent_hbm, rel_hbm, hidx_hbm, tidx_hbm, ridx_hbm, out_hbm,
             raw_h, raw_t, raw_r,
             ix_h_a, ix_t_a, ix_r_a, ix_h_b, ix_t_b, ix_r_b,
             bh_a, bt_a, br_a, bh_b, bt_b, br_b,
             stage, sem_a, sem_b):
  wid = lax.axis_index("s") * _NC + lax.axis_index("c")
  wbase = wid * _CPW

  pltpu.sync_copy(hidx_hbm.at[pl.ds(wbase, _CPW)], raw_h)
  pltpu.sync_copy(tidx_hbm.at[pl.ds(wbase, _CPW)], raw_t)
  pltpu.sync_copy(ridx_hbm.at[pl.ds(wbase, _CPW)], raw_r)

  iota = lax.broadcasted_iota(jnp.int32, (_L,), 0)

  def build_and_fire(s, ix_h, ix_t, ix_r, bh, bt, br, sem):
    base = s * _NT
    for c in range(_NT // _L):
      sl = pl.ds(c * _L, _L)
      bsl = pl.ds(base + c * _L, _L)
      ix_h[sl] = raw_h[bsl]
      ix_t[sl] = raw_t[bsl]
      ix_r[sl] = raw_r[bsl]
    for kk in range(_KK):
      pltpu.async_copy(ent_hbm.at[kk].at[ix_h], bh.at[kk], sem)
      pltpu.async_copy(ent_hbm.at[kk].at[ix_t], bt.at[kk], sem)
      pltpu.async_copy(rel_hbm.at[kk].at[ix_r], br.at[kk], sem)

  def compute(s, ix_h, ix_t, ix_r, bh, bt, br, sem):
    for kk in range(_KK):
      pltpu.make_async_copy(ent_hbm.at[kk].at[ix_h], bh.at[kk], sem).wait()
      pltpu.make_async_copy(ent_hbm.at[kk].at[ix_t], bt.at[kk], sem).wait()
      pltpu.make_async_copy(rel_hbm.at[kk].at[ix_r], br.at[kk], sem).wait()

    def slot_body(k, _):
      kkv = jnp.full((_L,), k // 2, jnp.int32)
      cb = (k % 2) * _DIM
      f = k % 4
      qcol = (s % 4) * _NT

      def group_loop(comb):
        def group_body(g, _):
          rows = g * _L + iota
          acc = jnp.zeros((_L,), jnp.float32)
          for dd in range(_DIM):
            col = cb + ((iota + dd) & (_DIM - 1))
            hv = plsc.load_gather(bh, [kkv, rows, col])
            tv = plsc.load_gather(bt, [kkv, rows, col])
            rv = plsc.load_gather(br, [kkv, rows, col])
            v = comb(hv, tv, rv)
            acc = acc + v * v
          stage[k, pl.ds(qcol + g * _L, _L)] = acc
          return 0
        lax.fori_loop(0, _NT // _L, group_body, 0, unroll=1)

      sh = jnp.where(f == 2, -1.0, 1.0)
      st = jnp.where(f == 0, -1.0, 1.0)
      sr = jnp.where(f == 1, -1.0, 1.0)

      @pl.when(f < 3)
      def _():
        group_loop(lambda h, t, r: sh * h + st * t + sr * r)

      @pl.when(f == 3)
      def _():
        group_loop(lambda h, t, r: h - r * t)

      return 0

    lax.fori_loop(0, _N_EMB, slot_body, 0, unroll=1)

    @pl.when(s % 4 == 3)
    def _():
      pltpu.sync_copy(
          stage, out_hbm.at[:, pl.ds(wbase + (s // 4) * 4 * _NT, 4 * _NT)])

  build_and_fire(0, ix_h_a, ix_t_a, ix_r_a, bh_a, bt_a, br_a, sem_a)

  def macro_body(m, _):
    s_a = 2 * m
    s_b = 2 * m + 1
    build_and_fire(s_b, ix_h_b, ix_t_b, ix_r_b, bh_b, bt_b, br_b, sem_b)
    compute(s_a, ix_h_a, ix_t_a, ix_r_a, bh_a, bt_a, br_a, sem_a)

    @pl.when(m < _NSTEP // 2 - 1)
    def _():
      build_and_fire(s_a + 2, ix_h_a, ix_t_a, ix_r_a, bh_a, bt_a, br_a,
                     sem_a)

    compute(s_b, ix_h_b, ix_t_b, ix_r_b, bh_b, bt_b, br_b, sem_b)
    return 0

  lax.fori_loop(0, _NSTEP // 2, macro_body, 0, unroll=1)


_sc_ssq = functools.partial(
    pl.kernel,
    out_type=jax.ShapeDtypeStruct((_N_EMB, _B2), jnp.float32),
    name="mde_ssq_gather",
    mesh=plsc.VectorSubcoreMesh(
        core_axis_name="c", subcore_axis_name="s",
        num_cores=_NC, num_subcores=_NS),
    compiler_params=pltpu.CompilerParams(
        needs_layout_passes=False, use_tc_tiling_on_sc=True),
    scratch_types=[
        pltpu.VMEM((_CPW,), jnp.int32),
        pltpu.VMEM((_CPW,), jnp.int32),
        pltpu.VMEM((_CPW,), jnp.int32),
        pltpu.VMEM((_NT,), jnp.int32),
        pltpu.VMEM((_NT,), jnp.int32),
        pltpu.VMEM((_NT,), jnp.int32),
        pltpu.VMEM((_NT,), jnp.int32),
        pltpu.VMEM((_NT,), jnp.int32),
        pltpu.VMEM((_NT,), jnp.int32),
        pltpu.VMEM((_KK, _NT, 2 * _DIM), jnp.float32),
        pltpu.VMEM((_KK, _NT, 2 * _DIM), jnp.float32),
        pltpu.VMEM((_KK, _NT, 2 * _DIM), jnp.float32),
        pltpu.VMEM((_KK, _NT, 2 * _DIM), jnp.float32),
        pltpu.VMEM((_KK, _NT, 2 * _DIM), jnp.float32),
        pltpu.VMEM((_KK, _NT, 2 * _DIM), jnp.float32),
        pltpu.VMEM((_N_EMB, 4 * _NT), jnp.float32),
        pltpu.SemaphoreType.DMA,
        pltpu.SemaphoreType.DMA,
    ],
)(_sc_body)


_W = (1.5 / 18.0, 3.0 / 18.0, 1.5 / 18.0, 3.0 / 18.0,
      1.5 / 18.0, 3.0 / 18.0, 1.5 / 18.0, 3.0 / 18.0)


def _tc_body(ssq_ref, loss_ref, pos_ref, neg_ref):
  n = jnp.sqrt(ssq_ref[...])
  score = _W[0] * n[0:1, :]
  for k in range(1, _N_EMB):
    score = score + _W[k] * n[k:k + 1, :]
  score = score - _PSI
  pos = jnp.sum(jnp.maximum(score[:, :_B] - (_LAMBDA_POS - _MARGIN), 0.0))
  neg = jnp.sum(jnp.maximum((_LAMBDA_NEG + _MARGIN) - score[:, _B:], 0.0))
  loss_ref[...] = jnp.full((1, 1), pos + neg, jnp.float32)
  pos_ref[...] = jnp.full((1, 1), pos, jnp.float32)
  neg_ref[...] = jnp.full((1, 1), neg, jnp.float32)


def _tc_finish(ssq):
  return pl.pallas_call(
      _tc_body,
      out_shape=(
          jax.ShapeDtypeStruct((1, 1), jnp.float32),
          jax.ShapeDtypeStruct((1, 1), jnp.float32),
          jax.ShapeDtypeStruct((1, 1), jnp.float32),
      ),
  )(ssq)


def kernel(x_train, x_train_negative, entity_emb, relation_emb):
  ent_t = jnp.transpose(entity_emb, (0, 2, 1))
  rel_t = jnp.transpose(relation_emb, (0, 2, 1))
  ent_p, rel_p = _repack(ent_t, rel_t)
  hidx = jnp.concatenate([x_train[:, 0], x_train_negative[:, 0]]).astype(jnp.int32)
  tidx = jnp.concatenate([x_train[:, 1], x_train_negative[:, 1]]).astype(jnp.int32)
  ridx = jnp.concatenate([x_train[:, 2], x_train_negative[:, 2]]).astype(jnp.int32)
  ssq = _sc_ssq(ent_p, rel_p, hidx, tidx, ridx)
  loss, pos, neg = _tc_finish(ssq)
  return (loss[0, 0], pos[0, 0], neg[0, 0])
